# baseline (speedup 1.0000x reference)
import functools

import jax
import jax.numpy as jnp
from jax import lax
from jax.experimental import pallas as pl
from jax.experimental.pallas import tpu as pltpu
from jax.experimental.pallas import tpu_sc as plsc
from jax._src.pallas import mpmd as _pl_mpmd

B = 128
N = 100000
BLKR = 16384
NBLK = (N + BLKR - 1) // BLKR

NTILES = 32
ZROWS = 256
NCHUNK = (N + ZROWS - 1) // ZROWS
ZLAST = (N - ZROWS) // 8 * 8
ZPT = (NCHUNK + NTILES - 1) // NTILES
OWN = 3200


CH = 4096
NFULL = N // CH
TAIL = N - NFULL * CH
NBUF = 8


def _tc_body(x_hbm, idx_ref, b0, b1, b2, b3, b4, b5, b6, b7, tbuf, iota_ref,
             s0, s1, s2, s3, s4, s5, s6, s7, st):
  bufs = (b0, b1, b2, b3, b4, b5, b6, b7)
  sems = (s0, s1, s2, s3, s4, s5, s6, s7)
  iota_ref[...] = lax.broadcasted_iota(jnp.int32, (CH, B), 0)

  copies = [
      pltpu.make_async_copy(
          x_hbm.at[pl.ds(c * CH, CH), :], bufs[c % NBUF], sems[c % NBUF])
      for c in range(NFULL)
  ]
  tail_copy = pltpu.make_async_copy(
      x_hbm.at[pl.ds(NFULL * CH, TAIL), :], tbuf, st)
  tail_copy.start()
  for c in range(NBUF):
    copies[c].start()

  maxv = jnp.full((1, B), -jnp.inf, jnp.float32)
  maxi = jnp.zeros((1, B), jnp.int32)

  def _scan(x, base, maxv, maxi, iota):
    bmax = jnp.max(x, axis=0, keepdims=True)
    lidx = jnp.min(jnp.where(x == bmax, iota, jnp.int32(N)),
                   axis=0, keepdims=True)
    better = bmax > maxv
    return (jnp.where(better, bmax, maxv),
            jnp.where(better, lidx + base, maxi))

  for c in range(NFULL):
    copies[c].wait()
    maxv, maxi = _scan(bufs[c % NBUF][...], c * CH, maxv, maxi,
                       iota_ref[...])
    if c + NBUF < NFULL:
      copies[c + NBUF].start()

  tail_copy.wait()
  maxv, maxi = _scan(tbuf[...], NFULL * CH, maxv, maxi,
                     iota_ref[pl.ds(0, TAIL), :])
  idx_ref[...] = maxi


_tc_call = pl.pallas_call(
    _tc_body,
    grid=(1,),
    in_specs=[pl.BlockSpec(memory_space=pltpu.MemorySpace.HBM)],
    out_specs=[pl.BlockSpec((1, B), lambda i: (0, 0))],
    out_shape=[jax.ShapeDtypeStruct((1, B), jnp.int32)],
    scratch_shapes=[
        pltpu.VMEM((CH, B), jnp.float32),
        pltpu.VMEM((CH, B), jnp.float32),
        pltpu.VMEM((CH, B), jnp.float32),
        pltpu.VMEM((CH, B), jnp.float32),
        pltpu.VMEM((CH, B), jnp.float32),
        pltpu.VMEM((CH, B), jnp.float32),
        pltpu.VMEM((CH, B), jnp.float32),
        pltpu.VMEM((CH, B), jnp.float32),
        pltpu.VMEM((TAIL, B), jnp.float32),
        pltpu.VMEM((CH, B), jnp.int32),
        pltpu.SemaphoreType.DMA,
        pltpu.SemaphoreType.DMA,
        pltpu.SemaphoreType.DMA,
        pltpu.SemaphoreType.DMA,
        pltpu.SemaphoreType.DMA,
        pltpu.SemaphoreType.DMA,
        pltpu.SemaphoreType.DMA,
        pltpu.SemaphoreType.DMA,
        pltpu.SemaphoreType.DMA,
    ],
)


def _sc_zero_body(out_hbm, zbuf, sem):
  wid = lax.axis_index("s") * 2 + lax.axis_index("c")

  def _zero(i, carry):
    for k in range(B // 16):
      zbuf[i, pl.ds(k * 16, 16)] = jnp.zeros((16,), jnp.float32)
    return carry
  lax.fori_loop(0, ZROWS, _zero, 0)

  copies = []
  for t in range(ZPT):
    c = wid + t * NTILES
    off = jnp.minimum(c * ZROWS, ZLAST)
    off = pl.multiple_of(off, 8)
    copies.append(pltpu.async_copy(
        zbuf, out_hbm.at[pl.ds(off, ZROWS), :], sem))
  for c in copies:
    c.wait()


def _sc_fix_body(zeros_hbm, idx_hbm, out_hbm, idx_v, tbuf, sem):
  del zeros_hbm
  del sem
  wid = lax.axis_index("s") * 2 + lax.axis_index("c")
  lo = wid * OWN
  hi = jnp.minimum(lo + OWN, N)
  pltpu.sync_copy(idx_hbm, idx_v)
  lanes = lax.broadcasted_iota(jnp.int32, (16,), 0)

  def _flush(r0):
    r0 = pl.multiple_of(lax.bitwise_and(r0, jnp.int32(-8)), 8)
    pltpu.sync_copy(tbuf, out_hbm.at[pl.ds(r0, 8), :])

  def _load(r0):
    r0 = pl.multiple_of(lax.bitwise_and(r0, jnp.int32(-8)), 8)
    pltpu.sync_copy(out_hbm.at[pl.ds(r0, 8), :], tbuf)

  cur = jnp.int32(-1)
  for j16 in range(0, B, 16):
    chunk = idx_v[0, pl.ds(j16, 16)]
    m = (chunk >= lo) & (chunk < hi)

    def _chunk(cur, chunk=chunk, m=m, j16=j16):
      chunk_f = chunk.astype(jnp.float32)
      for l in range(16):
        s_f = jnp.max(jnp.where(lanes == l, chunk_f, jnp.float32(-1.0)))
        s = s_f.astype(jnp.int32)
        hit = (s >= lo) & (s < hi)
        r0 = lax.bitwise_and(s, jnp.int32(-8))

        @pl.when(hit & (r0 != cur) & (cur >= 0))
        def _():
          _flush(cur)

        @pl.when(hit & (r0 != cur))
        def _():
          _load(r0)

        @pl.when(hit)
        def _():
          v = tbuf[s - r0, pl.ds(j16, 16)]
          tbuf[s - r0, pl.ds(j16, 16)] = jnp.where(lanes == l,
                                                   jnp.float32(1.0), v)
        cur = jnp.where(hit, r0, cur)
      return cur

    cur = lax.cond(jnp.any(m), _chunk, lambda cur: cur, cur)

  @pl.when(cur >= 0)
  def _():
    _flush(cur)


@functools.cache
def _get_sc_calls():
  mesh = plsc.VectorSubcoreMesh(core_axis_name="c", subcore_axis_name="s")
  params = pltpu.CompilerParams(needs_layout_passes=False)
  zero_call = _pl_mpmd._mpmd_map(
      [(mesh, _sc_zero_body)],
      jax.ShapeDtypeStruct((N, B), jnp.float32),
      compiler_params=params,
      scratch_types=[
          pltpu.VMEM((ZROWS, B), jnp.float32),
          pltpu.SemaphoreType.DMA,
      ],
  )
  fix_call = _pl_mpmd._mpmd_map(
      [(mesh, _sc_fix_body)],
      jax.ShapeDtypeStruct((N, B), jnp.float32),
      input_output_aliases={0: 0},
      compiler_params=params,
      scratch_types=[
          pltpu.VMEM((1, B), jnp.int32),
          pltpu.VMEM((8, B), jnp.float32),
          pltpu.SemaphoreType.DMA,
      ],
  )
  return zero_call, fix_call


def kernel(sample):
  zero_call, fix_call = _get_sc_calls()
  xt = sample.T
  idx = _tc_call(xt)[0]
  zeros = zero_call()
  out_t = fix_call(zeros, idx)
  return out_t.T


# speedup vs baseline: 1.0765x; 1.0765x over previous
import functools

import jax
import jax.numpy as jnp
from jax import lax
from jax.experimental import pallas as pl
from jax.experimental.pallas import tpu as pltpu
from jax.experimental.pallas import tpu_sc as plsc
from jax._src.pallas import mpmd as _pl_mpmd

B = 128
N = 100000
BLKR = 16384
NBLK = (N + BLKR - 1) // BLKR

NTILES = 32
ZROWS = 256
NCHUNK = (N + ZROWS - 1) // ZROWS
ZLAST = (N - ZROWS) // 8 * 8
ZPT = (NCHUNK + NTILES - 1) // NTILES
OWN = 3200


CH = 4096
NFULL = N // CH
TAIL = N - NFULL * CH
NBUF = 8


def _tc_body(x_hbm, idx_ref, b0, b1, b2, b3, b4, b5, b6, b7, tbuf, iota_ref,
             s0, s1, s2, s3, s4, s5, s6, s7, st):
  bufs = (b0, b1, b2, b3, b4, b5, b6, b7)
  sems = (s0, s1, s2, s3, s4, s5, s6, s7)
  iota_ref[...] = lax.broadcasted_iota(jnp.int32, (CH, B), 0)

  copies = [
      pltpu.make_async_copy(
          x_hbm.at[pl.ds(c * CH, CH), :], bufs[c % NBUF], sems[c % NBUF])
      for c in range(NFULL)
  ]
  tail_copy = pltpu.make_async_copy(
      x_hbm.at[pl.ds(NFULL * CH, TAIL), :], tbuf, st)
  tail_copy.start()
  for c in range(NBUF):
    copies[c].start()

  maxv = jnp.full((1, B), -jnp.inf, jnp.float32)
  maxi = jnp.zeros((1, B), jnp.int32)

  def _scan(x, base, maxv, maxi, iota):
    bmax = jnp.max(x, axis=0, keepdims=True)
    lidx = jnp.min(jnp.where(x == bmax, iota, jnp.int32(N)),
                   axis=0, keepdims=True)
    better = bmax > maxv
    return (jnp.where(better, bmax, maxv),
            jnp.where(better, lidx + base, maxi))

  for c in range(NFULL):
    copies[c].wait()
    maxv, maxi = _scan(bufs[c % NBUF][...], c * CH, maxv, maxi,
                       iota_ref[...])
    if c + NBUF < NFULL:
      copies[c + NBUF].start()

  tail_copy.wait()
  maxv, maxi = _scan(tbuf[...], NFULL * CH, maxv, maxi,
                     iota_ref[pl.ds(0, TAIL), :])
  idx_ref[...] = maxi


_tc_call = pl.pallas_call(
    _tc_body,
    grid=(1,),
    in_specs=[pl.BlockSpec(memory_space=pltpu.MemorySpace.HBM)],
    out_specs=[pl.BlockSpec((1, B), lambda i: (0, 0))],
    out_shape=[jax.ShapeDtypeStruct((1, B), jnp.int32)],
    scratch_shapes=[
        pltpu.VMEM((CH, B), jnp.float32),
        pltpu.VMEM((CH, B), jnp.float32),
        pltpu.VMEM((CH, B), jnp.float32),
        pltpu.VMEM((CH, B), jnp.float32),
        pltpu.VMEM((CH, B), jnp.float32),
        pltpu.VMEM((CH, B), jnp.float32),
        pltpu.VMEM((CH, B), jnp.float32),
        pltpu.VMEM((CH, B), jnp.float32),
        pltpu.VMEM((TAIL, B), jnp.float32),
        pltpu.VMEM((CH, B), jnp.int32),
        pltpu.SemaphoreType.DMA,
        pltpu.SemaphoreType.DMA,
        pltpu.SemaphoreType.DMA,
        pltpu.SemaphoreType.DMA,
        pltpu.SemaphoreType.DMA,
        pltpu.SemaphoreType.DMA,
        pltpu.SemaphoreType.DMA,
        pltpu.SemaphoreType.DMA,
        pltpu.SemaphoreType.DMA,
    ],
)


def _sc_zero_body(out_hbm, zbuf, sem):
  wid = lax.axis_index("s") * 2 + lax.axis_index("c")

  def _zero(i, carry):
    for k in range(B // 16):
      zbuf[i, pl.ds(k * 16, 16)] = jnp.zeros((16,), jnp.float32)
    return carry
  lax.fori_loop(0, ZROWS, _zero, 0)

  copies = []
  for t in range(ZPT):
    c = wid + t * NTILES
    off = jnp.minimum(c * ZROWS, ZLAST)
    off = pl.multiple_of(off, 8)
    copies.append(pltpu.async_copy(
        zbuf, out_hbm.at[pl.ds(off, ZROWS), :], sem))
  for c in copies:
    c.wait()


def _sc_fix_body(zeros_hbm, idx_hbm, out_hbm, idx_v, tbuf, sem):
  del zeros_hbm
  del sem
  wid = lax.axis_index("s") * 2 + lax.axis_index("c")
  lo = wid * OWN
  hi = jnp.minimum(lo + OWN, N)
  pltpu.sync_copy(idx_hbm, idx_v)
  lanes = lax.broadcasted_iota(jnp.int32, (16,), 0)

  def _flush(r0):
    r0 = pl.multiple_of(lax.bitwise_and(r0, jnp.int32(-8)), 8)
    pltpu.sync_copy(tbuf, out_hbm.at[pl.ds(r0, 8), :])

  def _load(r0):
    r0 = pl.multiple_of(lax.bitwise_and(r0, jnp.int32(-8)), 8)
    pltpu.sync_copy(out_hbm.at[pl.ds(r0, 8), :], tbuf)

  def _entry(j, cur):
    j16 = pl.multiple_of(j // 16 * 16, 16)
    chunk = idx_v[0, pl.ds(j16, 16)]
    s_f = jnp.max(jnp.where(lanes == j - j16, chunk.astype(jnp.float32),
                            jnp.float32(-1.0)))
    s = s_f.astype(jnp.int32)
    hit = (s >= lo) & (s < hi)
    r0 = lax.bitwise_and(s, jnp.int32(-8))

    @pl.when(hit & (r0 != cur) & (cur >= 0))
    def _():
      _flush(cur)

    @pl.when(hit & (r0 != cur))
    def _():
      _load(r0)

    @pl.when(hit)
    def _():
      l0 = pl.multiple_of(j16, 16)
      v = tbuf[s - r0, pl.ds(l0, 16)]
      tbuf[s - r0, pl.ds(l0, 16)] = jnp.where(lanes == j - j16,
                                              jnp.float32(1.0), v)
    return jnp.where(hit, r0, cur)

  last = lax.fori_loop(0, B, _entry, jnp.int32(-1))

  @pl.when(last >= 0)
  def _():
    _flush(last)


@functools.cache
def _get_sc_calls():
  mesh = plsc.VectorSubcoreMesh(core_axis_name="c", subcore_axis_name="s")
  params = pltpu.CompilerParams(needs_layout_passes=False)
  zero_call = _pl_mpmd._mpmd_map(
      [(mesh, _sc_zero_body)],
      jax.ShapeDtypeStruct((N, B), jnp.float32),
      compiler_params=params,
      scratch_types=[
          pltpu.VMEM((ZROWS, B), jnp.float32),
          pltpu.SemaphoreType.DMA,
      ],
  )
  fix_call = _pl_mpmd._mpmd_map(
      [(mesh, _sc_fix_body)],
      jax.ShapeDtypeStruct((N, B), jnp.float32),
      input_output_aliases={0: 0},
      compiler_params=params,
      scratch_types=[
          pltpu.VMEM((1, B), jnp.int32),
          pltpu.VMEM((8, B), jnp.float32),
          pltpu.SemaphoreType.DMA,
      ],
  )
  return zero_call, fix_call


def kernel(sample):
  zero_call, fix_call = _get_sc_calls()
  xt = sample.T
  idx = _tc_call(xt)[0]
  zeros = zero_call()
  out_t = fix_call(zeros, idx)
  return out_t.T


# speedup vs baseline: 1.0855x; 1.0083x over previous
import functools

import jax
import jax.numpy as jnp
from jax import lax
from jax.experimental import pallas as pl
from jax.experimental.pallas import tpu as pltpu
from jax.experimental.pallas import tpu_sc as plsc
from jax._src.pallas import mpmd as _pl_mpmd

B = 128
N = 100000
BLKR = 16384
NBLK = (N + BLKR - 1) // BLKR

NTILES = 32
ZROWS = 256
NCHUNK = (N + ZROWS - 1) // ZROWS
ZLAST = (N - ZROWS) // 8 * 8
ZPT = (NCHUNK + NTILES - 1) // NTILES
OWN = 3200


CH = 2048
NFULL = N // CH
TAIL = N - NFULL * CH
NBUF = 16


def _tc_body(x_hbm, idx_ref, bufs, tbuf, iota_ref, sems, st):
  iota_ref[...] = lax.broadcasted_iota(jnp.int32, (CH, B), 0)

  copies = [
      pltpu.make_async_copy(
          x_hbm.at[pl.ds(c * CH, CH), :], bufs.at[c % NBUF],
          sems.at[c % NBUF])
      for c in range(NFULL)
  ]
  tail_copy = pltpu.make_async_copy(
      x_hbm.at[pl.ds(NFULL * CH, TAIL), :], tbuf, st)
  tail_copy.start()
  for c in range(NBUF):
    copies[c].start()

  maxv = jnp.full((1, B), -jnp.inf, jnp.float32)
  maxi = jnp.zeros((1, B), jnp.int32)

  def _scan(x, base, maxv, maxi, iota):
    bmax = jnp.max(x, axis=0, keepdims=True)
    lidx = jnp.min(jnp.where(x == bmax, iota, jnp.int32(N)),
                   axis=0, keepdims=True)
    better = bmax > maxv
    return (jnp.where(better, bmax, maxv),
            jnp.where(better, lidx + base, maxi))

  for c in range(NFULL):
    copies[c].wait()
    maxv, maxi = _scan(bufs[c % NBUF], c * CH, maxv, maxi,
                       iota_ref[...])
    if c + NBUF < NFULL:
      copies[c + NBUF].start()

  tail_copy.wait()
  maxv, maxi = _scan(tbuf[...], NFULL * CH, maxv, maxi,
                     iota_ref[pl.ds(0, TAIL), :])
  idx_ref[...] = maxi


_tc_call = pl.pallas_call(
    _tc_body,
    grid=(1,),
    in_specs=[pl.BlockSpec(memory_space=pltpu.MemorySpace.HBM)],
    out_specs=[pl.BlockSpec((1, B), lambda i: (0, 0))],
    out_shape=[jax.ShapeDtypeStruct((1, B), jnp.int32)],
    scratch_shapes=[
        pltpu.VMEM((NBUF, CH, B), jnp.float32),
        pltpu.VMEM((TAIL, B), jnp.float32),
        pltpu.VMEM((CH, B), jnp.int32),
        pltpu.SemaphoreType.DMA((NBUF,)),
        pltpu.SemaphoreType.DMA,
    ],
)


def _sc_zero_body(out_hbm, zbuf, sem):
  wid = lax.axis_index("s") * 2 + lax.axis_index("c")

  def _zero(i, carry):
    for k in range(B // 16):
      zbuf[i, pl.ds(k * 16, 16)] = jnp.zeros((16,), jnp.float32)
    return carry
  lax.fori_loop(0, ZROWS, _zero, 0)

  copies = []
  for t in range(ZPT):
    c = wid + t * NTILES
    off = jnp.minimum(c * ZROWS, ZLAST)
    off = pl.multiple_of(off, 8)
    copies.append(pltpu.async_copy(
        zbuf, out_hbm.at[pl.ds(off, ZROWS), :], sem))
  for c in copies:
    c.wait()


def _sc_fix_body(zeros_hbm, idx_hbm, out_hbm, idx_v, tbuf, sem):
  del zeros_hbm
  del sem
  wid = lax.axis_index("s") * 2 + lax.axis_index("c")
  lo = wid * OWN
  hi = jnp.minimum(lo + OWN, N)
  pltpu.sync_copy(idx_hbm, idx_v)
  lanes = lax.broadcasted_iota(jnp.int32, (16,), 0)

  def _flush(r0):
    r0 = pl.multiple_of(lax.bitwise_and(r0, jnp.int32(-8)), 8)
    pltpu.sync_copy(tbuf, out_hbm.at[pl.ds(r0, 8), :])

  def _load(r0):
    r0 = pl.multiple_of(lax.bitwise_and(r0, jnp.int32(-8)), 8)
    pltpu.sync_copy(out_hbm.at[pl.ds(r0, 8), :], tbuf)

  def _entry(j, cur):
    j16 = pl.multiple_of(j // 16 * 16, 16)
    chunk = idx_v[0, pl.ds(j16, 16)]
    s_f = jnp.max(jnp.where(lanes == j - j16, chunk.astype(jnp.float32),
                            jnp.float32(-1.0)))
    s = s_f.astype(jnp.int32)
    hit = (s >= lo) & (s < hi)
    r0 = lax.bitwise_and(s, jnp.int32(-8))

    @pl.when(hit & (r0 != cur) & (cur >= 0))
    def _():
      _flush(cur)

    @pl.when(hit & (r0 != cur))
    def _():
      _load(r0)

    @pl.when(hit)
    def _():
      l0 = pl.multiple_of(j16, 16)
      v = tbuf[s - r0, pl.ds(l0, 16)]
      tbuf[s - r0, pl.ds(l0, 16)] = jnp.where(lanes == j - j16,
                                              jnp.float32(1.0), v)
    return jnp.where(hit, r0, cur)

  last = lax.fori_loop(0, B, _entry, jnp.int32(-1))

  @pl.when(last >= 0)
  def _():
    _flush(last)


@functools.cache
def _get_sc_calls():
  mesh = plsc.VectorSubcoreMesh(core_axis_name="c", subcore_axis_name="s")
  params = pltpu.CompilerParams(needs_layout_passes=False)
  zero_call = _pl_mpmd._mpmd_map(
      [(mesh, _sc_zero_body)],
      jax.ShapeDtypeStruct((N, B), jnp.float32),
      compiler_params=params,
      scratch_types=[
          pltpu.VMEM((ZROWS, B), jnp.float32),
          pltpu.SemaphoreType.DMA,
      ],
  )
  fix_call = _pl_mpmd._mpmd_map(
      [(mesh, _sc_fix_body)],
      jax.ShapeDtypeStruct((N, B), jnp.float32),
      input_output_aliases={0: 0},
      compiler_params=params,
      scratch_types=[
          pltpu.VMEM((1, B), jnp.int32),
          pltpu.VMEM((8, B), jnp.float32),
          pltpu.SemaphoreType.DMA,
      ],
  )
  return zero_call, fix_call


def kernel(sample):
  zero_call, fix_call = _get_sc_calls()
  xt = sample.T
  idx = _tc_call(xt)[0]
  zeros = zero_call()
  out_t = fix_call(zeros, idx)
  return out_t.T


# speedup vs baseline: 1.3126x; 1.2092x over previous
import functools

import jax
import jax.numpy as jnp
from jax import lax
from jax.experimental import pallas as pl
from jax.experimental.pallas import tpu as pltpu
from jax.experimental.pallas import tpu_sc as plsc
from jax._src.pallas import mpmd as _pl_mpmd

B = 128
N = 100000
BLKR = 16384
NBLK = (N + BLKR - 1) // BLKR

NTILES = 32
ZROWS = 256
NCHUNK = (N + ZROWS - 1) // ZROWS
ZLAST = (N - ZROWS) // 8 * 8
ZPT = (NCHUNK + NTILES - 1) // NTILES
OWN = 3200


CH = 2048
NFULL = N // CH
TAIL = N - NFULL * CH
NBUF = 16


def _tc_body(x_hbm, idx_ref, bufs, tbuf, iota_ref, sems, st):
  iota_ref[...] = lax.broadcasted_iota(jnp.int32, (CH, B), 0)

  copies = [
      pltpu.make_async_copy(
          x_hbm.at[pl.ds(c * CH, CH), :], bufs.at[c % NBUF],
          sems.at[c % NBUF])
      for c in range(NFULL)
  ]
  tail_copy = pltpu.make_async_copy(
      x_hbm.at[pl.ds(NFULL * CH, TAIL), :], tbuf, st)
  tail_copy.start()
  for c in range(NBUF):
    copies[c].start()

  maxv = jnp.full((1, B), -jnp.inf, jnp.float32)
  maxi = jnp.zeros((1, B), jnp.int32)

  def _scan(x, base, maxv, maxi, iota):
    bmax = jnp.max(x, axis=0, keepdims=True)
    lidx = jnp.min(jnp.where(x == bmax, iota, jnp.int32(N)),
                   axis=0, keepdims=True)
    better = bmax > maxv
    return (jnp.where(better, bmax, maxv),
            jnp.where(better, lidx + base, maxi))

  for c in range(NFULL):
    copies[c].wait()
    maxv, maxi = _scan(bufs[c % NBUF], c * CH, maxv, maxi,
                       iota_ref[...])
    if c + NBUF < NFULL:
      copies[c + NBUF].start()

  tail_copy.wait()
  maxv, maxi = _scan(tbuf[...], NFULL * CH, maxv, maxi,
                     iota_ref[pl.ds(0, TAIL), :])
  idx_ref[...] = maxi


_tc_call = pl.pallas_call(
    _tc_body,
    grid=(1,),
    in_specs=[pl.BlockSpec(memory_space=pltpu.MemorySpace.HBM)],
    out_specs=[pl.BlockSpec((1, B), lambda i: (0, 0))],
    out_shape=[jax.ShapeDtypeStruct((1, B), jnp.int32)],
    scratch_shapes=[
        pltpu.VMEM((NBUF, CH, B), jnp.float32),
        pltpu.VMEM((TAIL, B), jnp.float32),
        pltpu.VMEM((CH, B), jnp.int32),
        pltpu.SemaphoreType.DMA((NBUF,)),
        pltpu.SemaphoreType.DMA,
    ],
)


def _sc_zero_body(out_hbm, zbuf, sem):
  wid = lax.axis_index("s") * 2 + lax.axis_index("c")

  def _zero(i, carry):
    for k in range(B // 16):
      zbuf[i, pl.ds(k * 16, 16)] = jnp.zeros((16,), jnp.float32)
    return carry
  lax.fori_loop(0, ZROWS, _zero, 0)

  copies = []
  for t in range(ZPT):
    c = wid + t * NTILES
    off = jnp.minimum(c * ZROWS, ZLAST)
    off = pl.multiple_of(off, 8)
    copies.append(pltpu.async_copy(
        zbuf, out_hbm.at[pl.ds(off, ZROWS), :], sem))
  for c in copies:
    c.wait()


def _sc_fix_body(zeros_hbm, idx_hbm, out_hbm, idx_v, tbuf, sem):
  del zeros_hbm
  del sem
  wid = lax.axis_index("s") * 2 + lax.axis_index("c")
  lo = wid * OWN
  hi = jnp.minimum(lo + OWN, N)
  pltpu.sync_copy(idx_hbm, idx_v)
  lanes = lax.broadcasted_iota(jnp.int32, (16,), 0)

  def _flush(r0):
    r0 = pl.multiple_of(lax.bitwise_and(r0, jnp.int32(-8)), 8)
    pltpu.sync_copy(tbuf, out_hbm.at[pl.ds(r0, 8), :])

  def _load(r0):
    r0 = pl.multiple_of(lax.bitwise_and(r0, jnp.int32(-8)), 8)
    pltpu.sync_copy(out_hbm.at[pl.ds(r0, 8), :], tbuf)

  def _entry(j, cur):
    j16 = pl.multiple_of(j // 16 * 16, 16)
    chunk = idx_v[0, pl.ds(j16, 16)]
    s_f = jnp.max(jnp.where(lanes == j - j16, chunk.astype(jnp.float32),
                            jnp.float32(-1.0)))
    s = s_f.astype(jnp.int32)
    hit = (s >= lo) & (s < hi)
    r0 = lax.bitwise_and(s, jnp.int32(-8))

    @pl.when(hit & (r0 != cur) & (cur >= 0))
    def _():
      _flush(cur)

    @pl.when(hit & (r0 != cur))
    def _():
      _load(r0)

    @pl.when(hit)
    def _():
      l0 = pl.multiple_of(j16, 16)
      v = tbuf[s - r0, pl.ds(l0, 16)]
      tbuf[s - r0, pl.ds(l0, 16)] = jnp.where(lanes == j - j16,
                                              jnp.float32(1.0), v)
    return jnp.where(hit, r0, cur)

  last = lax.fori_loop(0, B, _entry, jnp.int32(-1))

  @pl.when(last >= 0)
  def _():
    _flush(last)


@functools.cache
def _get_sc_calls():
  mesh = plsc.VectorSubcoreMesh(core_axis_name="c", subcore_axis_name="s")
  params = pltpu.CompilerParams(needs_layout_passes=False)
  zero_call = _pl_mpmd._mpmd_map(
      [(mesh, _sc_zero_body)],
      jax.ShapeDtypeStruct((N, B), jnp.float32),
      compiler_params=params,
      scratch_types=[
          pltpu.VMEM((ZROWS, B), jnp.float32),
          pltpu.SemaphoreType.DMA,
      ],
  )
  fix_call = _pl_mpmd._mpmd_map(
      [(mesh, _sc_fix_body)],
      jax.ShapeDtypeStruct((N, B), jnp.float32),
      input_output_aliases={0: 0},
      compiler_params=params,
      scratch_types=[
          pltpu.VMEM((1, B), jnp.int32),
          pltpu.VMEM((8, B), jnp.float32),
          pltpu.SemaphoreType.DMA,
      ],
  )
  return zero_call, fix_call


def _tc_fix_body(zeros_hbm, idxs_ref, idxv_ref, out_hbm, bufs, rsem, wsem):
  del zeros_hbm
  reads = []
  for k in range(B):
    r0 = pl.multiple_of(lax.bitwise_and(idxs_ref[0, k], jnp.int32(-8)), 8)
    reads.append(pltpu.make_async_copy(
        out_hbm.at[pl.ds(r0, 8), :], bufs.at[k], rsem))
  for r in reads:
    r.start()

  idxv = idxv_ref[...]
  bvec = lax.shift_right_logical(idxv, 3)
  svec = lax.bitwise_and(idxv, jnp.int32(7))
  sub8 = lax.broadcasted_iota(jnp.int32, (8, B), 0)
  bvec8 = jnp.broadcast_to(bvec, (8, B))
  svec8 = jnp.broadcast_to(svec, (8, B))

  for k in range(B):
    reads[k].wait()
    bk = lax.shift_right_logical(idxs_ref[0, k], 3)
    patt = (bvec8 == bk) & (svec8 == sub8)
    bufs[k] = jnp.where(patt, jnp.float32(1.0), bufs[k])

  writes = []
  for k in range(B):
    r0 = pl.multiple_of(lax.bitwise_and(idxs_ref[0, k], jnp.int32(-8)), 8)
    writes.append(pltpu.make_async_copy(
        bufs.at[k], out_hbm.at[pl.ds(r0, 8), :], wsem))
  for w in writes:
    w.start()
  for w in writes:
    w.wait()


_tc_fix_call = pl.pallas_call(
    _tc_fix_body,
    grid=(1,),
    in_specs=[
        pl.BlockSpec(memory_space=pltpu.MemorySpace.HBM),
        pl.BlockSpec(memory_space=pltpu.MemorySpace.SMEM),
        pl.BlockSpec((1, B), lambda i: (0, 0)),
    ],
    out_specs=[pl.BlockSpec(memory_space=pltpu.MemorySpace.HBM)],
    out_shape=[jax.ShapeDtypeStruct((N, B), jnp.float32)],
    input_output_aliases={0: 0},
    scratch_shapes=[
        pltpu.VMEM((B, 8, B), jnp.float32),
        pltpu.SemaphoreType.DMA,
        pltpu.SemaphoreType.DMA,
    ],
)


def kernel(sample):
  zero_call, _ = _get_sc_calls()
  xt = sample.T
  idx = _tc_call(xt)[0]
  zeros = zero_call()
  out_t = _tc_fix_call(zeros, idx, idx)[0]
  return out_t.T


# speedup vs baseline: 1.3298x; 1.0131x over previous
import functools

import jax
import jax.numpy as jnp
from jax import lax
from jax.experimental import pallas as pl
from jax.experimental.pallas import tpu as pltpu
from jax.experimental.pallas import tpu_sc as plsc
from jax._src.pallas import mpmd as _pl_mpmd

B = 128
N = 100000

NTILES = 32
ZROWS = 256
NCHUNK = (N + ZROWS - 1) // ZROWS
ZLAST = (N - ZROWS) // 8 * 8
ZPT = (NCHUNK + NTILES - 1) // NTILES


CH = 2048
NFULL = N // CH
TAIL = N - NFULL * CH
NBUF = 16


def _tc_body(x_hbm, idx_ref, bufs, tbuf, iota_ref, sems, st):
  iota_ref[...] = lax.broadcasted_iota(jnp.int32, (CH, B), 0)

  copies = [
      pltpu.make_async_copy(
          x_hbm.at[pl.ds(c * CH, CH), :], bufs.at[c % NBUF],
          sems.at[c % NBUF])
      for c in range(NFULL)
  ]
  tail_copy = pltpu.make_async_copy(
      x_hbm.at[pl.ds(NFULL * CH, TAIL), :], tbuf, st)
  tail_copy.start()
  for c in range(NBUF):
    copies[c].start()

  maxv = jnp.full((1, B), -jnp.inf, jnp.float32)
  maxi = jnp.zeros((1, B), jnp.int32)

  def _scan(x, base, maxv, maxi, iota):
    bmax = jnp.max(x, axis=0, keepdims=True)
    lidx = jnp.min(jnp.where(x == bmax, iota, jnp.int32(N)),
                   axis=0, keepdims=True)
    better = bmax > maxv
    return (jnp.where(better, bmax, maxv),
            jnp.where(better, lidx + base, maxi))

  for c in range(NFULL):
    copies[c].wait()
    maxv, maxi = _scan(bufs[c % NBUF], c * CH, maxv, maxi,
                       iota_ref[...])
    if c + NBUF < NFULL:
      copies[c + NBUF].start()

  tail_copy.wait()
  maxv, maxi = _scan(tbuf[...], NFULL * CH, maxv, maxi,
                     iota_ref[pl.ds(0, TAIL), :])
  idx_ref[...] = maxi


_tc_call = pl.pallas_call(
    _tc_body,
    grid=(1,),
    in_specs=[pl.BlockSpec(memory_space=pltpu.MemorySpace.HBM)],
    out_specs=[pl.BlockSpec((1, B), lambda i: (0, 0))],
    out_shape=[jax.ShapeDtypeStruct((1, B), jnp.int32)],
    scratch_shapes=[
        pltpu.VMEM((NBUF, CH, B), jnp.float32),
        pltpu.VMEM((TAIL, B), jnp.float32),
        pltpu.VMEM((CH, B), jnp.int32),
        pltpu.SemaphoreType.DMA((NBUF,)),
        pltpu.SemaphoreType.DMA,
    ],
)


def _sc_zero_body(out_hbm, zbuf, sem):
  wid = lax.axis_index("s") * 2 + lax.axis_index("c")

  def _zero(i, carry):
    for k in range(B // 16):
      zbuf[i, pl.ds(k * 16, 16)] = jnp.zeros((16,), jnp.float32)
    return carry
  lax.fori_loop(0, ZROWS, _zero, 0)

  copies = []
  for t in range(ZPT):
    c = wid + t * NTILES
    off = jnp.minimum(c * ZROWS, ZLAST)
    off = pl.multiple_of(off, 8)
    copies.append(pltpu.async_copy(
        zbuf, out_hbm.at[pl.ds(off, ZROWS), :], sem))
  for c in copies:
    c.wait()


@functools.cache
def _get_zero_call():
  mesh = plsc.VectorSubcoreMesh(core_axis_name="c", subcore_axis_name="s")
  params = pltpu.CompilerParams(needs_layout_passes=False)
  zero_call = _pl_mpmd._mpmd_map(
      [(mesh, _sc_zero_body)],
      jax.ShapeDtypeStruct((N, B), jnp.float32),
      compiler_params=params,
      scratch_types=[
          pltpu.VMEM((ZROWS, B), jnp.float32),
          pltpu.SemaphoreType.DMA,
      ],
  )
  return zero_call


def _tc_fix_body(zeros_hbm, idxs_ref, idxv_ref, out_hbm, bufs, rsem, wsem):
  del zeros_hbm
  reads = []
  for k in range(B):
    r0 = pl.multiple_of(lax.bitwise_and(idxs_ref[0, k], jnp.int32(-8)), 8)
    reads.append(pltpu.make_async_copy(
        out_hbm.at[pl.ds(r0, 8), :], bufs.at[k], rsem))
  for r in reads:
    r.start()

  idxv = idxv_ref[...]
  bvec = lax.shift_right_logical(idxv, 3)
  svec = lax.bitwise_and(idxv, jnp.int32(7))
  sub8 = lax.broadcasted_iota(jnp.int32, (8, B), 0)
  bvec8 = jnp.broadcast_to(bvec, (8, B))
  svec8 = jnp.broadcast_to(svec, (8, B))

  for k in range(B):
    reads[k].wait()
    bk = lax.shift_right_logical(idxs_ref[0, k], 3)
    patt = (bvec8 == bk) & (svec8 == sub8)
    bufs[k] = jnp.where(patt, jnp.float32(1.0), bufs[k])

  writes = []
  for k in range(B):
    r0 = pl.multiple_of(lax.bitwise_and(idxs_ref[0, k], jnp.int32(-8)), 8)
    writes.append(pltpu.make_async_copy(
        bufs.at[k], out_hbm.at[pl.ds(r0, 8), :], wsem))
  for w in writes:
    w.start()
  for w in writes:
    w.wait()


_tc_fix_call = pl.pallas_call(
    _tc_fix_body,
    grid=(1,),
    in_specs=[
        pl.BlockSpec(memory_space=pltpu.MemorySpace.HBM),
        pl.BlockSpec(memory_space=pltpu.MemorySpace.SMEM),
        pl.BlockSpec((1, B), lambda i: (0, 0)),
    ],
    out_specs=[pl.BlockSpec(memory_space=pltpu.MemorySpace.HBM)],
    out_shape=[jax.ShapeDtypeStruct((N, B), jnp.float32)],
    input_output_aliases={0: 0},
    scratch_shapes=[
        pltpu.VMEM((B, 8, B), jnp.float32),
        pltpu.SemaphoreType.DMA,
        pltpu.SemaphoreType.DMA,
    ],
)


def kernel(sample):
  zero_call = _get_zero_call()
  xt = sample.T
  idx = _tc_call(xt)[0]
  zeros = zero_call()
  out_t = _tc_fix_call(zeros, idx, idx)[0]
  return out_t.T
